# Initial kernel scaffold; baseline (speedup 1.0000x reference)
import jax
import jax.numpy as jnp
from jax.experimental import pallas as pl


def kernel(feature, edge_index, W1, b1, W2, b2, Wfc, bfc):
    raise NotImplementedError("write your pallas kernel here")


# speedup vs baseline: 2.3975x; 2.3975x over previous
import functools

import jax
import jax.numpy as jnp
from jax import lax
from jax.experimental import pallas as pl
from jax.experimental.pallas import tpu as pltpu
from jax.experimental.pallas import tpu_sc as plsc

L = 16
N_REAL = 19
N_PAD = 32
E_REAL = 342
E_TOT = E_REAL + N_REAL
E_PAD = 368
EV = E_PAD // L
NV = N_PAD // L
ACC = L * N_PAD
COLS = 1600
COLS_PAD = 2048
CPS = 64
CV = CPS // L

_MESH = plsc.VectorSubcoreMesh(core_axis_name="c", subcore_axis_name="s")


_LOG2E2 = 2.885390081777927
_LN2 = 0.6931471805599453
_EXP2_C = (1.0, 0.6931471805599453, 0.2402265069591007, 0.05550410866482158,
           0.009618129107628477, 0.0013333558146428443, 0.00015403530393381608,
           1.525273380405984e-05, 1.3215486790144307e-06)


def _tanh16(v):
    t = jnp.clip(v * _LOG2E2, -30.0, 30.0)
    k = t.astype(jnp.int32)
    f = t - k.astype(jnp.float32)
    p = jnp.full((L,), _EXP2_C[8], jnp.float32)
    for i in range(7, -1, -1):
        p = p * f + _EXP2_C[i]
    e = p * plsc.bitcast((k + 127) << 23, jnp.float32)
    return (e - 1.0) / (e + 1.0)


def _rsqrt16(v):
    i = plsc.bitcast(v, jnp.int32)
    magic = jnp.full((L,), 0x5F3759DF, jnp.int32)
    y = plsc.bitcast(magic - (i >> 1), jnp.float32)
    half = 0.5 * v
    for _ in range(3):
        y = y * (1.5 - half * y * y)
    return y


def _bcast(ref, j):
    return plsc.load_gather(ref, [jnp.full((L,), j, jnp.int32)])


def _zero_acc(acc_v):
    z = jnp.zeros((L,), jnp.float32)
    for i in range(ACC // L):
        acc_v[pl.ds(i * L, L)] = z


def _reduce_acc(acc_v, out_ref):
    for i in range(NV):
        s = acc_v[pl.ds(i * L, L)]
        for l in range(1, L):
            s = s + acc_v[pl.ds(l * N_PAD + i * L, L)]
        out_ref[pl.ds(i * L, L)] = s


@functools.partial(
    pl.kernel,
    out_type=jax.ShapeDtypeStruct((COLS_PAD,), jnp.float32),
    mesh=_MESH,
    compiler_params=pltpu.CompilerParams(needs_layout_passes=False),
    scratch_types=[
        pltpu.VMEM((E_PAD,), jnp.int32),
        pltpu.VMEM((E_PAD,), jnp.int32),
        pltpu.VMEM((N_PAD,), jnp.float32),
        pltpu.VMEM((10 * L,), jnp.float32),
        pltpu.VMEM((N_PAD,), jnp.float32),
        pltpu.VMEM((E_PAD,), jnp.float32),
        pltpu.VMEM((N_PAD,), jnp.float32),
        pltpu.VMEM((ACC,), jnp.float32),
        pltpu.VMEM((N_PAD,), jnp.float32),
        pltpu.VMEM((N_PAD,), jnp.float32),
        pltpu.VMEM((CPS * N_PAD,), jnp.float32),
        pltpu.VMEM((CPS,), jnp.float32),
        pltpu.VMEM((CPS,), jnp.float32),
    ],
)
def _sc_model(src_hbm, dst_hbm, feat_hbm, par_hbm, wfc_hbm, bfc_hbm, out_hbm,
              src_v, dst_v, feat_v, par_v, dinv_v, norm_v, hpre_v,
              acc_v, agg_v, h2_v, wfc_v, bfc_v, out_v):
    wid = lax.axis_index("s") * _MESH.num_cores + lax.axis_index("c")

    pltpu.sync_copy(src_hbm, src_v)
    pltpu.sync_copy(dst_hbm, dst_v)
    pltpu.sync_copy(feat_hbm, feat_v)
    pltpu.sync_copy(par_hbm, par_v)
    pltpu.sync_copy(wfc_hbm.at[wid], wfc_v)
    pltpu.sync_copy(bfc_hbm.at[pl.ds(wid * CPS, CPS)], bfc_v)

    ones = jnp.ones((L,), jnp.float32)

    _zero_acc(acc_v)
    for e in range(EV):
        plsc.addupdate_scatter(acc_v, [dst_v[pl.ds(e * L, L)]], ones)
    _reduce_acc(acc_v, dinv_v)
    for i in range(NV):
        d = dinv_v[pl.ds(i * L, L)]
        dinv_v[pl.ds(i * L, L)] = _rsqrt16(d)

    nmask = jnp.full((L,), N_PAD - 1, jnp.int32)
    for e in range(EV):
        s = src_v[pl.ds(e * L, L)]
        d = dst_v[pl.ds(e * L, L)] & nmask
        norm_v[pl.ds(e * L, L)] = (plsc.load_gather(dinv_v, [s]) *
                                   plsc.load_gather(dinv_v, [d]))

    _zero_acc(acc_v)
    for e in range(EV):
        s = src_v[pl.ds(e * L, L)]
        m = plsc.load_gather(feat_v, [s]) * norm_v[pl.ds(e * L, L)]
        plsc.addupdate_scatter(acc_v, [dst_v[pl.ds(e * L, L)]], m)
    _reduce_acc(acc_v, agg_v)

    for i in range(NV):
        a = agg_v[pl.ds(i * L, L)]
        acc = jnp.zeros((L,), jnp.float32)
        for k in range(3):
            w1k = par_v[pl.ds(k * L, L)]
            b1k = par_v[pl.ds((3 + k) * L, L)]
            w2k = par_v[pl.ds((6 + k) * L, L)]
            acc = acc + _tanh16(w1k * a + b1k) * w2k
        hpre_v[pl.ds(i * L, L)] = acc

    _zero_acc(acc_v)
    for e in range(EV):
        s = src_v[pl.ds(e * L, L)]
        m = plsc.load_gather(hpre_v, [s]) * norm_v[pl.ds(e * L, L)]
        plsc.addupdate_scatter(acc_v, [dst_v[pl.ds(e * L, L)]], m)
    _reduce_acc(acc_v, agg_v)
    b2 = par_v[pl.ds(9 * L, L)]
    for i in range(NV):
        h2_v[pl.ds(i * L, L)] = _tanh16(agg_v[pl.ds(i * L, L)] + b2)

    acc = [bfc_v[pl.ds(c * L, L)] for c in range(CV)]
    for n in range(1, N_REAL + 1):
        hb = _bcast(h2_v, n)
        for c in range(CV):
            acc[c] = acc[c] + hb * wfc_v[pl.ds(n * CPS + c * L, L)]
    for c in range(CV):
        out_v[pl.ds(c * L, L)] = acc[c]
    pltpu.sync_copy(out_v, out_hbm.at[pl.ds(wid * CPS, CPS)])


def kernel(feature, edge_index, W1, b1, W2, b2, Wfc, bfc):
    ei = edge_index.astype(jnp.int32) + 1
    loop = jnp.arange(1, N_REAL + 1, dtype=jnp.int32)
    pad = jnp.zeros((E_PAD - E_TOT,), jnp.int32)
    src = jnp.concatenate([ei[0], loop, pad])
    dst = jnp.concatenate([ei[1], loop, pad])
    lane = jnp.tile(jnp.arange(L, dtype=jnp.int32), E_PAD // L)
    dst_off = dst + lane * N_PAD

    feat = jnp.zeros((N_PAD,), jnp.float32).at[1:N_REAL + 1].set(feature[:, 0])
    scal = jnp.concatenate([W1[0], b1, W2[:, 0], b2])
    par = jnp.repeat(scal.astype(jnp.float32), L)

    wfc_pad = (jnp.zeros((N_PAD, COLS_PAD), jnp.float32)
               .at[1:N_REAL + 1, :COLS].set(Wfc))
    slabs = (wfc_pad.reshape(N_PAD, COLS_PAD // CPS, CPS)
             .transpose(1, 0, 2).reshape(COLS_PAD // CPS, N_PAD * CPS))
    bfc_pad = jnp.zeros((COLS_PAD,), jnp.float32).at[:COLS].set(bfc)

    out = _sc_model(src, dst_off, feat, par, slabs, bfc_pad)
    return out[:COLS]


# speedup vs baseline: 2.7009x; 1.1266x over previous
import functools

import jax
import jax.numpy as jnp
from jax import lax
from jax.experimental import pallas as pl
from jax.experimental.pallas import tpu as pltpu
from jax.experimental.pallas import tpu_sc as plsc

L = 16
N_REAL = 19
N_PAD = 32
E_REAL = 342
E_PAD = 352
EV = E_PAD // L
NV = N_PAD // L
ACC = L * N_PAD
COLS = 1600
COLS_PAD = 2048
CPS = 64
CV = CPS // L

PI_SRC = 0
PI_DST = E_PAD
PI_LEN = 2 * E_PAD
PF_FEAT = 0
PF_PAR = N_PAD
PF_BFC = N_PAD + 10 * L
PF_LEN = N_PAD + 10 * L + COLS_PAD

_MESH = plsc.VectorSubcoreMesh(core_axis_name="c", subcore_axis_name="s")

_LOG2E2 = 2.885390081777927
_EXP2_C = (1.0, 0.6931471805599453, 0.2402265069591007, 0.05550410866482158,
           0.009618129107628477, 0.0013333558146428443, 0.00015403530393381608,
           1.525273380405984e-05, 1.3215486790144307e-06)


def _tanh16(v):
    t = jnp.clip(v * _LOG2E2, -30.0, 30.0)
    k = t.astype(jnp.int32)
    f = t - k.astype(jnp.float32)
    p = jnp.full((L,), _EXP2_C[8], jnp.float32)
    for i in range(7, -1, -1):
        p = p * f + _EXP2_C[i]
    e = p * plsc.bitcast((k + 127) << 23, jnp.float32)
    return (e - 1.0) / (e + 1.0)


def _rsqrt16(v):
    i = plsc.bitcast(v, jnp.int32)
    magic = jnp.full((L,), 0x5F3759DF, jnp.int32)
    y = plsc.bitcast(magic - (i >> 1), jnp.float32)
    half = 0.5 * v
    for _ in range(3):
        y = y * (1.5 - half * y * y)
    return y


def _bcast(ref, j):
    return plsc.load_gather(ref, [jnp.full((L,), j, jnp.int32)])


def _zero_acc(acc_v):
    z = jnp.zeros((L,), jnp.float32)
    for i in range(ACC // L):
        acc_v[pl.ds(i * L, L)] = z


def _reduce_acc(acc_v, out_ref):
    for i in range(NV):
        s = acc_v[pl.ds(i * L, L)]
        for l in range(1, L):
            s = s + acc_v[pl.ds(l * N_PAD + i * L, L)]
        out_ref[pl.ds(i * L, L)] = s


@functools.partial(
    pl.kernel,
    out_type=jax.ShapeDtypeStruct((COLS_PAD,), jnp.float32),
    mesh=_MESH,
    compiler_params=pltpu.CompilerParams(needs_layout_passes=False),
    scratch_types=[
        pltpu.VMEM((PI_LEN,), jnp.int32),
        pltpu.VMEM((PF_LEN,), jnp.float32),
        pltpu.VMEM((N_PAD,), jnp.float32),
        pltpu.VMEM((E_PAD,), jnp.float32),
        pltpu.VMEM((N_PAD,), jnp.float32),
        pltpu.VMEM((ACC,), jnp.float32),
        pltpu.VMEM((N_PAD,), jnp.float32),
        pltpu.VMEM((N_PAD,), jnp.float32),
        pltpu.VMEM((CPS * N_PAD,), jnp.float32),
        pltpu.VMEM((CPS,), jnp.float32),
        pltpu.SemaphoreType.DMA,
        pltpu.SemaphoreType.DMA,
        pltpu.SemaphoreType.DMA,
    ],
)
def _sc_model(pi_hbm, pf_hbm, wfc_hbm, out_hbm,
              pk_i, pk_f, dinv_v, norm_v, hpre_v, acc_v, agg_v, h2_v,
              wfc_v, out_v, sem_i, sem_f, sem_w):
    wid = lax.axis_index("s") * _MESH.num_cores + lax.axis_index("c")

    cp_i = pltpu.async_copy(pi_hbm, pk_i, sem_i)
    cp_f = pltpu.async_copy(pf_hbm, pk_f, sem_f)
    cp_w = pltpu.async_copy(wfc_hbm.at[wid], wfc_v, sem_w)

    ones = jnp.ones((L,), jnp.float32)

    _zero_acc(acc_v)
    cp_i.wait()
    for e in range(EV):
        plsc.addupdate_scatter(acc_v, [pk_i[pl.ds(PI_DST + e * L, L)]], ones)
    _reduce_acc(acc_v, dinv_v)
    for i in range(NV):
        d = dinv_v[pl.ds(i * L, L)] + ones
        dinv_v[pl.ds(i * L, L)] = _rsqrt16(d)

    nmask = jnp.full((L,), N_PAD - 1, jnp.int32)
    cp_f.wait()

    _zero_acc(acc_v)
    for e in range(EV):
        s = pk_i[pl.ds(PI_SRC + e * L, L)]
        doff = pk_i[pl.ds(PI_DST + e * L, L)]
        nrm = (plsc.load_gather(dinv_v, [s]) *
               plsc.load_gather(dinv_v, [doff & nmask]))
        norm_v[pl.ds(e * L, L)] = nrm
        m = plsc.load_gather(pk_f, [s]) * nrm
        plsc.addupdate_scatter(acc_v, [doff], m)
    _reduce_acc(acc_v, agg_v)
    for i in range(NV):
        dv = dinv_v[pl.ds(i * L, L)]
        agg_v[pl.ds(i * L, L)] = (agg_v[pl.ds(i * L, L)] +
                                  pk_f[pl.ds(PF_FEAT + i * L, L)] * dv * dv)

    for i in range(NV):
        a = agg_v[pl.ds(i * L, L)]
        acc = jnp.zeros((L,), jnp.float32)
        for k in range(3):
            w1k = pk_f[pl.ds(PF_PAR + k * L, L)]
            b1k = pk_f[pl.ds(PF_PAR + (3 + k) * L, L)]
            w2k = pk_f[pl.ds(PF_PAR + (6 + k) * L, L)]
            acc = acc + _tanh16(w1k * a + b1k) * w2k
        hpre_v[pl.ds(i * L, L)] = acc

    _zero_acc(acc_v)
    for e in range(EV):
        s = pk_i[pl.ds(PI_SRC + e * L, L)]
        m = plsc.load_gather(hpre_v, [s]) * norm_v[pl.ds(e * L, L)]
        plsc.addupdate_scatter(acc_v, [pk_i[pl.ds(PI_DST + e * L, L)]], m)
    _reduce_acc(acc_v, agg_v)
    b2 = pk_f[pl.ds(PF_PAR + 9 * L, L)]
    for i in range(NV):
        dv = dinv_v[pl.ds(i * L, L)]
        a = agg_v[pl.ds(i * L, L)] + hpre_v[pl.ds(i * L, L)] * dv * dv
        h2_v[pl.ds(i * L, L)] = _tanh16(a + b2)

    cp_w.wait()
    acc = [pk_f[pl.ds(PF_BFC + wid * CPS + c * L, L)] for c in range(CV)]
    for n in range(1, N_REAL + 1):
        hb = _bcast(h2_v, n)
        for c in range(CV):
            acc[c] = acc[c] + hb * wfc_v[pl.ds(n * CPS + c * L, L)]
    for c in range(CV):
        out_v[pl.ds(c * L, L)] = acc[c]
    pltpu.sync_copy(out_v, out_hbm.at[pl.ds(wid * CPS, CPS)])


def kernel(feature, edge_index, W1, b1, W2, b2, Wfc, bfc):
    ei = edge_index.astype(jnp.int32) + 1
    src = jnp.zeros((E_PAD,), jnp.int32).at[:E_REAL].set(ei[0])
    dst = jnp.zeros((E_PAD,), jnp.int32).at[:E_REAL].set(ei[1])
    lane = jnp.tile(jnp.arange(L, dtype=jnp.int32), EV)
    pk_i = jnp.concatenate([src, dst + lane * N_PAD])

    feat = jnp.zeros((N_PAD,), jnp.float32).at[1:N_REAL + 1].set(feature[:, 0])
    scal = jnp.concatenate([W1[0], b1, W2[:, 0], b2])
    par = jnp.repeat(scal.astype(jnp.float32), L)
    bfc_pad = jnp.zeros((COLS_PAD,), jnp.float32).at[:COLS].set(bfc)
    pk_f = jnp.concatenate([feat, par, bfc_pad])

    wfc_pad = (jnp.zeros((N_PAD, COLS_PAD), jnp.float32)
               .at[1:N_REAL + 1, :COLS].set(Wfc))
    slabs = (wfc_pad.reshape(N_PAD, COLS_PAD // CPS, CPS)
             .transpose(1, 0, 2).reshape(COLS_PAD // CPS, N_PAD * CPS))

    out = _sc_model(pk_i, pk_f, slabs)
    return out[:COLS]


# speedup vs baseline: 2.9723x; 1.1005x over previous
import functools

import jax
import jax.numpy as jnp
from jax import lax
from jax.experimental import pallas as pl
from jax.experimental.pallas import tpu as pltpu
from jax.experimental.pallas import tpu_sc as plsc

L = 16
N_REAL = 19
N_PAD = 32
E_REAL = 342
E_PAD = 352
EV = E_PAD // L
NV = N_PAD // L
ACC = L * N_PAD
COLS = 1600
CPS = 64
CV = CPS // L
NFC = COLS // CPS

_MESH = plsc.VectorSubcoreMesh(core_axis_name="c", subcore_axis_name="s")

_LOG2E2 = 2.885390081777927
_EXP2_C = (1.0, 0.6931471805599453, 0.2402265069591007, 0.05550410866482158,
           0.009618129107628477, 0.0013333558146428443, 0.00015403530393381608,
           1.525273380405984e-05, 1.3215486790144307e-06)


def _tanh16(v):
    t = jnp.clip(v * _LOG2E2, -30.0, 30.0)
    k = t.astype(jnp.int32)
    f = t - k.astype(jnp.float32)
    p = jnp.full((L,), _EXP2_C[8], jnp.float32)
    for i in range(7, -1, -1):
        p = p * f + _EXP2_C[i]
    e = p * plsc.bitcast((k + 127) << 23, jnp.float32)
    return (e - 1.0) / (e + 1.0)


def _rsqrt16(v):
    i = plsc.bitcast(v, jnp.int32)
    magic = jnp.full((L,), 0x5F3759DF, jnp.int32)
    y = plsc.bitcast(magic - (i >> 1), jnp.float32)
    half = 0.5 * v
    for _ in range(3):
        y = y * (1.5 - half * y * y)
    return y


def _bcast(ref, j):
    return plsc.load_gather(ref, [jnp.full((L,), j, jnp.int32)])


def _zero_acc(acc_v):
    z = jnp.zeros((L,), jnp.float32)
    for i in range(ACC // L):
        acc_v[pl.ds(i * L, L)] = z


def _reduce_acc(acc_v, out_ref):
    for i in range(NV):
        s = acc_v[pl.ds(i * L, L)]
        for l in range(1, L):
            s = s + acc_v[pl.ds(l * N_PAD + i * L, L)]
        out_ref[pl.ds(i * L, L)] = s


@functools.partial(
    pl.kernel,
    out_type=jax.ShapeDtypeStruct((COLS,), jnp.float32),
    mesh=_MESH,
    compiler_params=pltpu.CompilerParams(needs_layout_passes=False),
    scratch_types=[
        pltpu.VMEM((2 * E_REAL,), jnp.int32),
        pltpu.VMEM((E_PAD,), jnp.int32),
        pltpu.VMEM((N_REAL,), jnp.float32),
        pltpu.VMEM((L,), jnp.float32),
        pltpu.VMEM((N_PAD,), jnp.float32),
        pltpu.VMEM((E_PAD,), jnp.float32),
        pltpu.VMEM((N_PAD,), jnp.float32),
        pltpu.VMEM((ACC,), jnp.float32),
        pltpu.VMEM((N_PAD,), jnp.float32),
        pltpu.VMEM((N_PAD,), jnp.float32),
        pltpu.VMEM((N_REAL * CPS,), jnp.float32),
        pltpu.VMEM((CPS,), jnp.float32),
        pltpu.VMEM((CPS,), jnp.float32),
        pltpu.SemaphoreType.DMA,
        pltpu.SemaphoreType.DMA,
        pltpu.SemaphoreType.DMA,
    ],
)
def _sc_model(ei_hbm, feat_hbm, scal_hbm, wfc_hbm, bfc_hbm, out_hbm,
              ei_v, dst_v, feat_v, scal_v, dinv_v, norm_v, hpre_v,
              acc_v, agg_v, h2_v, wfc_v, bfc_v, out_v,
              sem_e, sem_p, sem_w):
    wid = lax.axis_index("s") * _MESH.num_cores + lax.axis_index("c")
    do_fc = wid < NFC

    cp_e = pltpu.async_copy(ei_hbm, ei_v, sem_e)
    cp_f = pltpu.async_copy(feat_hbm, feat_v, sem_p)
    cp_s = pltpu.async_copy(scal_hbm, scal_v, sem_p)

    @pl.when(do_fc)
    def _():
        pltpu.async_copy(bfc_hbm.at[pl.ds(wid * CPS, CPS)], bfc_v, sem_w)
        for r in range(N_REAL):
            pltpu.async_copy(wfc_hbm.at[pl.ds(r * COLS + wid * CPS, CPS)],
                             wfc_v.at[pl.ds(r * CPS, CPS)], sem_w)

    ones = jnp.ones((L,), jnp.float32)
    ione = jnp.full((L,), 1, jnp.int32)
    iota = lax.iota(jnp.int32, L)
    lane32 = iota * N_PAD

    _zero_acc(acc_v)
    cp_e.wait()
    for e in range(EV):
        idx = iota + (E_REAL + e * L)
        if (e + 1) * L <= E_REAL:
            d = plsc.load_gather(ei_v, [idx]) + ione
        else:
            valid = iota < (E_REAL - e * L)
            d = plsc.load_gather(ei_v, [jnp.minimum(idx, 2 * E_REAL - 1)])
            d = jnp.where(valid, d + ione, 0)
        doff = d + lane32
        dst_v[pl.ds(e * L, L)] = doff
        plsc.addupdate_scatter(acc_v, [doff], ones)
    _reduce_acc(acc_v, dinv_v)
    for i in range(NV):
        d = dinv_v[pl.ds(i * L, L)] + ones
        dinv_v[pl.ds(i * L, L)] = _rsqrt16(d)

    nmask = jnp.full((L,), N_PAD - 1, jnp.int32)
    cp_f.wait()
    cp_s.wait()

    _zero_acc(acc_v)
    for e in range(EV):
        s = ei_v[pl.ds(e * L, L)]
        doff = dst_v[pl.ds(e * L, L)]
        nrm = (plsc.load_gather(dinv_v, [s + ione]) *
               plsc.load_gather(dinv_v, [doff & nmask]))
        norm_v[pl.ds(e * L, L)] = nrm
        m = plsc.load_gather(feat_v, [s]) * nrm
        plsc.addupdate_scatter(acc_v, [doff], m)
    _reduce_acc(acc_v, agg_v)
    for i in range(NV):
        dv = dinv_v[pl.ds(i * L, L)]
        fshift = plsc.load_gather(
            feat_v, [jnp.clip(iota + (i * L - 1), 0, N_REAL - 1)])
        fshift = jnp.where((iota + i * L >= 1) & (iota + i * L <= N_REAL), fshift, 0.0)
        agg_v[pl.ds(i * L, L)] = agg_v[pl.ds(i * L, L)] + fshift * dv * dv

    for i in range(NV):
        a = agg_v[pl.ds(i * L, L)]
        acc = jnp.zeros((L,), jnp.float32)
        for k in range(3):
            w1k = _bcast(scal_v, 1 + k)
            b1k = _bcast(scal_v, 4 + k)
            w2k = _bcast(scal_v, 7 + k)
            acc = acc + _tanh16(w1k * a + b1k) * w2k
        hpre_v[pl.ds(i * L, L)] = acc

    _zero_acc(acc_v)
    for e in range(EV):
        s = ei_v[pl.ds(e * L, L)]
        m = plsc.load_gather(hpre_v, [s + ione]) * norm_v[pl.ds(e * L, L)]
        plsc.addupdate_scatter(acc_v, [dst_v[pl.ds(e * L, L)]], m)
    _reduce_acc(acc_v, agg_v)
    b2 = _bcast(scal_v, 10)
    for i in range(NV):
        dv = dinv_v[pl.ds(i * L, L)]
        a = agg_v[pl.ds(i * L, L)] + hpre_v[pl.ds(i * L, L)] * dv * dv
        h2_v[pl.ds(i * L, L)] = _tanh16(a + b2)

    @pl.when(do_fc)
    def _():
        pltpu.make_async_copy(bfc_hbm.at[pl.ds(wid * CPS, CPS)], bfc_v,
                              sem_w).wait()
        for r in range(N_REAL):
            pltpu.make_async_copy(
                wfc_hbm.at[pl.ds(r * COLS + wid * CPS, CPS)],
                wfc_v.at[pl.ds(r * CPS, CPS)], sem_w).wait()
        acc = [bfc_v[pl.ds(c * L, L)] for c in range(CV)]
        for n in range(1, N_REAL + 1):
            hb = _bcast(h2_v, n)
            for c in range(CV):
                acc[c] = acc[c] + hb * wfc_v[pl.ds((n - 1) * CPS + c * L, L)]
        for c in range(CV):
            out_v[pl.ds(c * L, L)] = acc[c]
        pltpu.sync_copy(out_v, out_hbm.at[pl.ds(wid * CPS, CPS)])


def kernel(feature, edge_index, W1, b1, W2, b2, Wfc, bfc):
    ei_flat = edge_index.astype(jnp.int32).reshape(2 * E_REAL)
    feat = feature.reshape(N_REAL)
    scal = jnp.concatenate([
        jnp.zeros((1,), jnp.float32), W1[0], b1, W2[:, 0], b2,
        jnp.zeros((L - 11,), jnp.float32)])
    wfc_flat = Wfc.reshape(N_REAL * COLS)
    return _sc_model(ei_flat, feat, scal, wfc_flat, bfc)


# speedup vs baseline: 3.0563x; 1.0283x over previous
import functools

import jax
import jax.numpy as jnp
from jax import lax
from jax.experimental import pallas as pl
from jax.experimental.pallas import tpu as pltpu
from jax.experimental.pallas import tpu_sc as plsc

L = 16
N_REAL = 19
N_PAD = 32
E_REAL = 342
E_PAD = 352
EV = E_PAD // L
NV = N_PAD // L
ACC = L * N_PAD
COLS = 1600
CPS = 64
CV = CPS // L
NFC = COLS // CPS

_MESH = plsc.VectorSubcoreMesh(core_axis_name="c", subcore_axis_name="s")

_LOG2E2 = 2.885390081777927
_EXP2_C = (1.0, 0.6931471805599453, 0.2402265069591007, 0.05550410866482158,
           0.009618129107628477, 0.0013333558146428443, 0.00015403530393381608,
           1.525273380405984e-05, 1.3215486790144307e-06)


def _tanh16(v):
    t = jnp.clip(v * _LOG2E2, -30.0, 30.0)
    k = t.astype(jnp.int32)
    f = t - k.astype(jnp.float32)
    p = jnp.full((L,), _EXP2_C[8], jnp.float32)
    for i in range(7, -1, -1):
        p = p * f + _EXP2_C[i]
    e = p * plsc.bitcast((k + 127) << 23, jnp.float32)
    return (e - 1.0) / (e + 1.0)


def _rsqrt16(v):
    i = plsc.bitcast(v, jnp.int32)
    magic = jnp.full((L,), 0x5F3759DF, jnp.int32)
    y = plsc.bitcast(magic - (i >> 1), jnp.float32)
    half = 0.5 * v
    for _ in range(3):
        y = y * (1.5 - half * y * y)
    return y


def _bcast(ref, j):
    return plsc.load_gather(ref, [jnp.full((L,), j, jnp.int32)])


def _zero_acc(acc_v):
    z = jnp.zeros((L,), jnp.float32)
    for i in range(ACC // L):
        acc_v[pl.ds(i * L, L)] = z


def _reduce_acc(acc_v, out_ref):
    for i in range(NV):
        s = acc_v[pl.ds(i * L, L)]
        for l in range(1, L):
            s = s + acc_v[pl.ds(l * N_PAD + i * L, L)]
        out_ref[pl.ds(i * L, L)] = s


@functools.partial(
    pl.kernel,
    out_type=jax.ShapeDtypeStruct((COLS,), jnp.float32),
    mesh=_MESH,
    compiler_params=pltpu.CompilerParams(needs_layout_passes=False),
    scratch_types=[
        pltpu.VMEM((E_REAL,), jnp.int32),
        pltpu.VMEM((E_REAL,), jnp.int32),
        pltpu.VMEM((E_PAD,), jnp.int32),
        pltpu.VMEM((30,), jnp.float32),
        pltpu.VMEM((N_PAD,), jnp.float32),
        pltpu.VMEM((E_PAD,), jnp.float32),
        pltpu.VMEM((N_PAD,), jnp.float32),
        pltpu.VMEM((ACC,), jnp.float32),
        pltpu.VMEM((N_PAD,), jnp.float32),
        pltpu.VMEM((N_PAD,), jnp.float32),
        pltpu.VMEM((N_REAL * CPS,), jnp.float32),
        pltpu.VMEM((CPS,), jnp.float32),
        pltpu.VMEM((CPS,), jnp.float32),
        pltpu.SemaphoreType.DMA,
        pltpu.SemaphoreType.DMA,
        pltpu.SemaphoreType.DMA,
    ],
)
def _sc_model(ei_hbm, sf_hbm, wfc_hbm, bfc_hbm, out_hbm,
              srcd_v, dstraw_v, dst_v, sf_v, dinv_v, norm_v, hpre_v,
              acc_v, agg_v, h2_v, wfc_v, bfc_v, out_v,
              sem_e, sem_p, sem_w):
    wid = lax.axis_index("s") * _MESH.num_cores + lax.axis_index("c")
    do_fc = wid < NFC

    cp_s0 = pltpu.async_copy(ei_hbm.at[0], srcd_v, sem_e)
    cp_d0 = pltpu.async_copy(ei_hbm.at[1], dstraw_v, sem_e)
    cp_f = pltpu.async_copy(sf_hbm, sf_v, sem_p)

    @pl.when(do_fc)
    def _():
        pltpu.async_copy(bfc_hbm.at[pl.ds(wid * CPS, CPS)], bfc_v, sem_w)
        for r in range(N_REAL):
            pltpu.async_copy(wfc_hbm.at[r, pl.ds(wid * CPS, CPS)],
                             wfc_v.at[pl.ds(r * CPS, CPS)], sem_w)

    ones = jnp.ones((L,), jnp.float32)
    ione = jnp.full((L,), 1, jnp.int32)
    iota = lax.iota(jnp.int32, L)
    lane32 = iota * N_PAD

    _zero_acc(acc_v)
    cp_s0.wait()
    cp_d0.wait()
    for e in range(EV):
        idx = iota + e * L
        if (e + 1) * L <= E_REAL:
            d = plsc.load_gather(dstraw_v, [idx]) + ione
        else:
            valid = iota < (E_REAL - e * L)
            d = plsc.load_gather(dstraw_v, [jnp.minimum(idx, E_REAL - 1)])
            d = jnp.where(valid, d + ione, 0)
        doff = d + lane32
        dst_v[pl.ds(e * L, L)] = doff
        plsc.addupdate_scatter(acc_v, [doff], ones)
    _reduce_acc(acc_v, dinv_v)
    for i in range(NV):
        d = dinv_v[pl.ds(i * L, L)] + ones
        dinv_v[pl.ds(i * L, L)] = _rsqrt16(d)

    nmask = jnp.full((L,), N_PAD - 1, jnp.int32)
    ifeat = jnp.full((L,), 11, jnp.int32)
    cp_f.wait()

    _zero_acc(acc_v)
    for e in range(EV):
        if (e + 1) * L <= E_REAL:
            s = srcd_v[pl.ds(e * L, L)]
        else:
            s = plsc.load_gather(
                srcd_v, [jnp.minimum(iota + e * L, E_REAL - 1)])
            s = jnp.where(iota < (E_REAL - e * L), s, 0)
        doff = dst_v[pl.ds(e * L, L)]
        nrm = (plsc.load_gather(dinv_v, [s + ione]) *
               plsc.load_gather(dinv_v, [doff & nmask]))
        norm_v[pl.ds(e * L, L)] = nrm
        m = plsc.load_gather(sf_v, [s + ifeat]) * nrm
        plsc.addupdate_scatter(acc_v, [doff], m)
    _reduce_acc(acc_v, agg_v)
    for i in range(NV):
        dv = dinv_v[pl.ds(i * L, L)]
        fshift = plsc.load_gather(
            sf_v, [jnp.clip(iota + (i * L - 1), 0, N_REAL - 1) + ifeat])
        fshift = jnp.where((iota + i * L >= 1) & (iota + i * L <= N_REAL), fshift, 0.0)
        agg_v[pl.ds(i * L, L)] = agg_v[pl.ds(i * L, L)] + fshift * dv * dv

    for i in range(NV):
        a = agg_v[pl.ds(i * L, L)]
        acc = jnp.zeros((L,), jnp.float32)
        for k in range(3):
            w1k = _bcast(sf_v, 1 + k)
            b1k = _bcast(sf_v, 4 + k)
            w2k = _bcast(sf_v, 7 + k)
            acc = acc + _tanh16(w1k * a + b1k) * w2k
        hpre_v[pl.ds(i * L, L)] = acc

    _zero_acc(acc_v)
    for e in range(EV):
        if (e + 1) * L <= E_REAL:
            s = srcd_v[pl.ds(e * L, L)]
        else:
            s = plsc.load_gather(
                srcd_v, [jnp.minimum(iota + e * L, E_REAL - 1)])
            s = jnp.where(iota < (E_REAL - e * L), s, 0)
        m = plsc.load_gather(hpre_v, [s + ione]) * norm_v[pl.ds(e * L, L)]
        plsc.addupdate_scatter(acc_v, [dst_v[pl.ds(e * L, L)]], m)
    _reduce_acc(acc_v, agg_v)
    b2 = _bcast(sf_v, 10)
    for i in range(NV):
        dv = dinv_v[pl.ds(i * L, L)]
        a = agg_v[pl.ds(i * L, L)] + hpre_v[pl.ds(i * L, L)] * dv * dv
        h2_v[pl.ds(i * L, L)] = _tanh16(a + b2)

    @pl.when(do_fc)
    def _():
        pltpu.make_async_copy(bfc_hbm.at[pl.ds(wid * CPS, CPS)], bfc_v,
                              sem_w).wait()
        for r in range(N_REAL):
            pltpu.make_async_copy(
                wfc_hbm.at[r, pl.ds(wid * CPS, CPS)],
                wfc_v.at[pl.ds(r * CPS, CPS)], sem_w).wait()
        acc = [bfc_v[pl.ds(c * L, L)] for c in range(CV)]
        for n in range(1, N_REAL + 1):
            hb = _bcast(h2_v, n)
            for c in range(CV):
                acc[c] = acc[c] + hb * wfc_v[pl.ds((n - 1) * CPS + c * L, L)]
        for c in range(CV):
            out_v[pl.ds(c * L, L)] = acc[c]
        pltpu.sync_copy(out_v, out_hbm.at[pl.ds(wid * CPS, CPS)])


def kernel(feature, edge_index, W1, b1, W2, b2, Wfc, bfc):
    sf = jnp.concatenate([
        jnp.zeros((1,), jnp.float32), W1[0], b1, W2[:, 0], b2,
        feature[:, 0]])
    return _sc_model(edge_index.astype(jnp.int32), sf, Wfc, bfc)


# speedup vs baseline: 3.1382x; 1.0268x over previous
import functools

import jax
import jax.numpy as jnp
from jax import lax
from jax.experimental import pallas as pl
from jax.experimental.pallas import tpu as pltpu
from jax.experimental.pallas import tpu_sc as plsc

L = 16
N_REAL = 19
N_PAD = 32
E_REAL = 342
E_PAD = 352
EV = E_PAD // L
NV = N_PAD // L
ACC = L * N_PAD
COLS = 1600
CPS = 64
CV = CPS // L
NFC = COLS // CPS

_MESH = plsc.VectorSubcoreMesh(core_axis_name="c", subcore_axis_name="s")

_LOG2E2 = 2.885390081777927
_EXP2_C = (1.0, 0.6931471805599453, 0.2402265069591007, 0.05550410866482158,
           0.009618129107628477, 0.0013333558146428443, 0.00015403530393381608,
           1.525273380405984e-05, 1.3215486790144307e-06)


def _tanh16(v):
    t = jnp.clip(v * _LOG2E2, -30.0, 30.0)
    k = t.astype(jnp.int32)
    f = t - k.astype(jnp.float32)
    p = jnp.full((L,), _EXP2_C[8], jnp.float32)
    for i in range(7, -1, -1):
        p = p * f + _EXP2_C[i]
    e = p * plsc.bitcast((k + 127) << 23, jnp.float32)
    return (e - 1.0) / (e + 1.0)


def _rsqrt16(v):
    i = plsc.bitcast(v, jnp.int32)
    magic = jnp.full((L,), 0x5F3759DF, jnp.int32)
    y = plsc.bitcast(magic - (i >> 1), jnp.float32)
    half = 0.5 * v
    for _ in range(3):
        y = y * (1.5 - half * y * y)
    return y


def _bcast(ref, j):
    return plsc.load_gather(ref, [jnp.full((L,), j, jnp.int32)])


def _zero_acc(acc_v):
    z = jnp.zeros((L,), jnp.float32)
    for i in range(ACC // L):
        acc_v[pl.ds(i * L, L)] = z


def _reduce_acc(acc_v, out_ref):
    for i in range(NV):
        vals = [acc_v[pl.ds(l * N_PAD + i * L, L)] for l in range(L)]
        while len(vals) > 1:
            vals = [vals[j] + vals[j + 1] for j in range(0, len(vals), 2)]
        out_ref[pl.ds(i * L, L)] = vals[0]


@functools.partial(
    pl.kernel,
    out_type=jax.ShapeDtypeStruct((COLS,), jnp.float32),
    mesh=_MESH,
    compiler_params=pltpu.CompilerParams(needs_layout_passes=False),
    scratch_types=[
        pltpu.VMEM((E_REAL,), jnp.int32),
        pltpu.VMEM((E_REAL,), jnp.int32),
        pltpu.VMEM((E_PAD,), jnp.int32),
        pltpu.VMEM((30,), jnp.float32),
        pltpu.VMEM((N_PAD,), jnp.float32),
        pltpu.VMEM((N_PAD,), jnp.float32),
        pltpu.VMEM((N_PAD,), jnp.float32),
        pltpu.VMEM((ACC,), jnp.float32),
        pltpu.VMEM((N_PAD,), jnp.float32),
        pltpu.VMEM((N_PAD,), jnp.float32),
        pltpu.VMEM((N_REAL * CPS,), jnp.float32),
        pltpu.VMEM((CPS,), jnp.float32),
        pltpu.VMEM((CPS,), jnp.float32),
        pltpu.SemaphoreType.DMA,
        pltpu.SemaphoreType.DMA,
        pltpu.SemaphoreType.DMA,
    ],
)
def _sc_model(ei_hbm, sf_hbm, wfc_hbm, bfc_hbm, out_hbm,
              srcd_v, dstraw_v, dst_v, sf_v, dinv_v, fs_v, hpre_v,
              acc_v, agg_v, h2_v, wfc_v, bfc_v, out_v,
              sem_e, sem_p, sem_w):
    wid = lax.axis_index("s") * _MESH.num_cores + lax.axis_index("c")
    do_fc = wid < NFC

    cp_s0 = pltpu.async_copy(ei_hbm.at[0], srcd_v, sem_e)
    cp_d0 = pltpu.async_copy(ei_hbm.at[1], dstraw_v, sem_e)
    cp_f = pltpu.async_copy(sf_hbm, sf_v, sem_p)

    @pl.when(do_fc)
    def _():
        pltpu.async_copy(bfc_hbm.at[pl.ds(wid * CPS, CPS)], bfc_v, sem_w)
        for r in range(N_REAL):
            pltpu.async_copy(wfc_hbm.at[r, pl.ds(wid * CPS, CPS)],
                             wfc_v.at[pl.ds(r * CPS, CPS)], sem_w)

    ones = jnp.ones((L,), jnp.float32)
    ione = jnp.full((L,), 1, jnp.int32)
    iota = lax.iota(jnp.int32, L)
    lane32 = iota * N_PAD

    _zero_acc(acc_v)
    cp_s0.wait()
    cp_d0.wait()
    for e in range(EV):
        idx = iota + e * L
        if (e + 1) * L <= E_REAL:
            d = plsc.load_gather(dstraw_v, [idx]) + ione
        else:
            valid = iota < (E_REAL - e * L)
            d = plsc.load_gather(dstraw_v, [jnp.minimum(idx, E_REAL - 1)])
            d = jnp.where(valid, d + ione, 0)
        doff = d + lane32
        dst_v[pl.ds(e * L, L)] = doff
        plsc.addupdate_scatter(acc_v, [doff], ones)
    _reduce_acc(acc_v, dinv_v)
    for i in range(NV):
        d = dinv_v[pl.ds(i * L, L)] + ones
        dinv_v[pl.ds(i * L, L)] = _rsqrt16(d)

    ifeat = jnp.full((L,), 11, jnp.int32)
    cp_f.wait()

    for i in range(NV):
        dv = dinv_v[pl.ds(i * L, L)]
        fshift = plsc.load_gather(
            sf_v, [jnp.clip(iota + (i * L - 1), 0, N_REAL - 1) + ifeat])
        fshift = jnp.where(
            (iota + i * L >= 1) & (iota + i * L <= N_REAL), fshift, 0.0)
        fs_v[pl.ds(i * L, L)] = fshift * dv

    _zero_acc(acc_v)
    for e in range(EV):
        if (e + 1) * L <= E_REAL:
            s = srcd_v[pl.ds(e * L, L)]
        else:
            s = plsc.load_gather(
                srcd_v, [jnp.minimum(iota + e * L, E_REAL - 1)])
            s = jnp.where(iota < (E_REAL - e * L), s, 0)
        m = plsc.load_gather(fs_v, [s + ione])
        plsc.addupdate_scatter(acc_v, [dst_v[pl.ds(e * L, L)]], m)
    _reduce_acc(acc_v, agg_v)
    for i in range(NV):
        dv = dinv_v[pl.ds(i * L, L)]
        agg_v[pl.ds(i * L, L)] = dv * (agg_v[pl.ds(i * L, L)] +
                                       fs_v[pl.ds(i * L, L)])

    for i in range(NV):
        a = agg_v[pl.ds(i * L, L)]
        acc = jnp.zeros((L,), jnp.float32)
        for k in range(3):
            w1k = _bcast(sf_v, 1 + k)
            b1k = _bcast(sf_v, 4 + k)
            w2k = _bcast(sf_v, 7 + k)
            acc = acc + _tanh16(w1k * a + b1k) * w2k
        hpre_v[pl.ds(i * L, L)] = acc

    for i in range(NV):
        fs_v[pl.ds(i * L, L)] = (hpre_v[pl.ds(i * L, L)] *
                                 dinv_v[pl.ds(i * L, L)])
    _zero_acc(acc_v)
    for e in range(EV):
        if (e + 1) * L <= E_REAL:
            s = srcd_v[pl.ds(e * L, L)]
        else:
            s = plsc.load_gather(
                srcd_v, [jnp.minimum(iota + e * L, E_REAL - 1)])
            s = jnp.where(iota < (E_REAL - e * L), s, 0)
        m = plsc.load_gather(fs_v, [s + ione])
        plsc.addupdate_scatter(acc_v, [dst_v[pl.ds(e * L, L)]], m)
    _reduce_acc(acc_v, agg_v)
    b2 = _bcast(sf_v, 10)
    for i in range(NV):
        dv = dinv_v[pl.ds(i * L, L)]
        a = dv * (agg_v[pl.ds(i * L, L)] + fs_v[pl.ds(i * L, L)])
        h2_v[pl.ds(i * L, L)] = _tanh16(a + b2)

    @pl.when(do_fc)
    def _():
        pltpu.make_async_copy(bfc_hbm.at[pl.ds(wid * CPS, CPS)], bfc_v,
                              sem_w).wait()
        for r in range(N_REAL):
            pltpu.make_async_copy(
                wfc_hbm.at[r, pl.ds(wid * CPS, CPS)],
                wfc_v.at[pl.ds(r * CPS, CPS)], sem_w).wait()
        acc = [bfc_v[pl.ds(c * L, L)] for c in range(CV)]
        for n in range(1, N_REAL + 1):
            hb = _bcast(h2_v, n)
            for c in range(CV):
                acc[c] = acc[c] + hb * wfc_v[pl.ds((n - 1) * CPS + c * L, L)]
        for c in range(CV):
            out_v[pl.ds(c * L, L)] = acc[c]
        pltpu.sync_copy(out_v, out_hbm.at[pl.ds(wid * CPS, CPS)])


def kernel(feature, edge_index, W1, b1, W2, b2, Wfc, bfc):
    sf = jnp.concatenate([
        jnp.zeros((1,), jnp.float32), W1[0], b1, W2[:, 0], b2,
        feature[:, 0]])
    return _sc_model(edge_index.astype(jnp.int32), sf, Wfc, bfc)


# speedup vs baseline: 3.2167x; 1.0250x over previous
import functools

import jax
import jax.numpy as jnp
from jax import lax
from jax.experimental import pallas as pl
from jax.experimental.pallas import tpu as pltpu
from jax.experimental.pallas import tpu_sc as plsc

L = 16
N_REAL = 19
N_PAD = 32
E_REAL = 342
E_PAD = 352
EV = E_PAD // L
NV = N_PAD // L
ACC = L * N_PAD
COLS = 1600
CPS = 64
CV = CPS // L
NFC = COLS // CPS

_MESH = plsc.VectorSubcoreMesh(core_axis_name="c", subcore_axis_name="s")

_LOG2E2 = 2.885390081777927
_EXP2_C = (1.0, 0.6931471805599453, 0.2402265069591007, 0.05550410866482158,
           0.009618129107628477, 0.0013333558146428443, 0.00015403530393381608,
           1.525273380405984e-05, 1.3215486790144307e-06)


def _tanh16(v):
    t = jnp.clip(v * _LOG2E2, -30.0, 30.0)
    k = t.astype(jnp.int32)
    f = t - k.astype(jnp.float32)
    p = jnp.full((L,), _EXP2_C[8], jnp.float32)
    for i in range(7, -1, -1):
        p = p * f + _EXP2_C[i]
    e = p * plsc.bitcast((k + 127) << 23, jnp.float32)
    return (e - 1.0) / (e + 1.0)


def _rsqrt16(v):
    i = plsc.bitcast(v, jnp.int32)
    magic = jnp.full((L,), 0x5F3759DF, jnp.int32)
    y = plsc.bitcast(magic - (i >> 1), jnp.float32)
    half = 0.5 * v
    for _ in range(3):
        y = y * (1.5 - half * y * y)
    return y


def _bcast(ref, j):
    return plsc.load_gather(ref, [jnp.full((L,), j, jnp.int32)])


def _zero_acc(acc_v):
    z = jnp.zeros((L,), jnp.float32)

    def _zbody(i, carry):
        acc_v[pl.ds(i * L, L)] = z
        return carry

    lax.fori_loop(0, ACC // L, _zbody, 0, unroll=False)


def _edge_pass(srcd_v, dst_v, fs_v, acc_v, iota, ione):
    def _body(e, carry):
        s = srcd_v[pl.ds(e * L, L)]
        m = plsc.load_gather(fs_v, [s + ione])
        plsc.addupdate_scatter(acc_v, [dst_v[pl.ds(e * L, L)]], m)
        return carry

    lax.fori_loop(0, EV - 1, _body, 0, unroll=False)
    e = EV - 1
    s = plsc.load_gather(srcd_v, [jnp.minimum(iota + e * L, E_REAL - 1)])
    s = jnp.where(iota < (E_REAL - e * L), s, 0)
    m = plsc.load_gather(fs_v, [s + ione])
    plsc.addupdate_scatter(acc_v, [dst_v[pl.ds(e * L, L)]], m)


def _reduce_acc(acc_v, out_ref):
    for i in range(NV):
        vals = [acc_v[pl.ds(l * N_PAD + i * L, L)] for l in range(L)]
        while len(vals) > 1:
            vals = [vals[j] + vals[j + 1] for j in range(0, len(vals), 2)]
        out_ref[pl.ds(i * L, L)] = vals[0]


@functools.partial(
    pl.kernel,
    out_type=jax.ShapeDtypeStruct((COLS,), jnp.float32),
    mesh=_MESH,
    compiler_params=pltpu.CompilerParams(needs_layout_passes=False),
    scratch_types=[
        pltpu.VMEM((E_REAL,), jnp.int32),
        pltpu.VMEM((E_REAL,), jnp.int32),
        pltpu.VMEM((E_PAD,), jnp.int32),
        pltpu.VMEM((30,), jnp.float32),
        pltpu.VMEM((N_PAD,), jnp.float32),
        pltpu.VMEM((N_PAD,), jnp.float32),
        pltpu.VMEM((N_PAD,), jnp.float32),
        pltpu.VMEM((ACC,), jnp.float32),
        pltpu.VMEM((N_PAD,), jnp.float32),
        pltpu.VMEM((N_PAD,), jnp.float32),
        pltpu.VMEM((N_REAL * CPS,), jnp.float32),
        pltpu.VMEM((CPS,), jnp.float32),
        pltpu.VMEM((CPS,), jnp.float32),
        pltpu.SemaphoreType.DMA,
        pltpu.SemaphoreType.DMA,
        pltpu.SemaphoreType.DMA,
    ],
)
def _sc_model(ei_hbm, sf_hbm, wfc_hbm, bfc_hbm, out_hbm,
              srcd_v, dstraw_v, dst_v, sf_v, dinv_v, fs_v, hpre_v,
              acc_v, agg_v, h2_v, wfc_v, bfc_v, out_v,
              sem_e, sem_p, sem_w):
    wid = lax.axis_index("s") * _MESH.num_cores + lax.axis_index("c")
    do_fc = wid < NFC

    cp_s0 = pltpu.async_copy(ei_hbm.at[0], srcd_v, sem_e)
    cp_d0 = pltpu.async_copy(ei_hbm.at[1], dstraw_v, sem_e)
    cp_f = pltpu.async_copy(sf_hbm, sf_v, sem_p)

    @pl.when(do_fc)
    def _():
        pltpu.async_copy(bfc_hbm.at[pl.ds(wid * CPS, CPS)], bfc_v, sem_w)
        for r in range(N_REAL):
            pltpu.async_copy(wfc_hbm.at[r, pl.ds(wid * CPS, CPS)],
                             wfc_v.at[pl.ds(r * CPS, CPS)], sem_w)

    ones = jnp.ones((L,), jnp.float32)
    ione = jnp.full((L,), 1, jnp.int32)
    iota = lax.iota(jnp.int32, L)
    lane32 = iota * N_PAD

    _zero_acc(acc_v)
    cp_s0.wait()
    cp_d0.wait()

    def _repack_body(e, carry):
        idx = iota + e * L
        d = plsc.load_gather(dstraw_v, [idx]) + ione
        doff = d + lane32
        dst_v[pl.ds(e * L, L)] = doff
        plsc.addupdate_scatter(acc_v, [doff], ones)
        return carry

    lax.fori_loop(0, EV - 1, _repack_body, 0, unroll=False)
    e = EV - 1
    idx = iota + e * L
    valid = iota < (E_REAL - e * L)
    d = plsc.load_gather(dstraw_v, [jnp.minimum(idx, E_REAL - 1)])
    d = jnp.where(valid, d + ione, 0)
    doff = d + lane32
    dst_v[pl.ds(e * L, L)] = doff
    plsc.addupdate_scatter(acc_v, [doff], ones)
    _reduce_acc(acc_v, dinv_v)
    for i in range(NV):
        d = dinv_v[pl.ds(i * L, L)] + ones
        dinv_v[pl.ds(i * L, L)] = _rsqrt16(d)

    ifeat = jnp.full((L,), 11, jnp.int32)
    cp_f.wait()

    for i in range(NV):
        dv = dinv_v[pl.ds(i * L, L)]
        fshift = plsc.load_gather(
            sf_v, [jnp.clip(iota + (i * L - 1), 0, N_REAL - 1) + ifeat])
        fshift = jnp.where(
            (iota + i * L >= 1) & (iota + i * L <= N_REAL), fshift, 0.0)
        fs_v[pl.ds(i * L, L)] = fshift * dv

    _zero_acc(acc_v)
    _edge_pass(srcd_v, dst_v, fs_v, acc_v, iota, ione)
    _reduce_acc(acc_v, agg_v)
    for i in range(NV):
        dv = dinv_v[pl.ds(i * L, L)]
        agg_v[pl.ds(i * L, L)] = dv * (agg_v[pl.ds(i * L, L)] +
                                       fs_v[pl.ds(i * L, L)])

    for i in range(NV):
        a = agg_v[pl.ds(i * L, L)]
        acc = jnp.zeros((L,), jnp.float32)
        for k in range(3):
            w1k = _bcast(sf_v, 1 + k)
            b1k = _bcast(sf_v, 4 + k)
            w2k = _bcast(sf_v, 7 + k)
            acc = acc + _tanh16(w1k * a + b1k) * w2k
        hpre_v[pl.ds(i * L, L)] = acc

    for i in range(NV):
        fs_v[pl.ds(i * L, L)] = (hpre_v[pl.ds(i * L, L)] *
                                 dinv_v[pl.ds(i * L, L)])
    _zero_acc(acc_v)
    _edge_pass(srcd_v, dst_v, fs_v, acc_v, iota, ione)
    _reduce_acc(acc_v, agg_v)
    b2 = _bcast(sf_v, 10)
    for i in range(NV):
        dv = dinv_v[pl.ds(i * L, L)]
        a = dv * (agg_v[pl.ds(i * L, L)] + fs_v[pl.ds(i * L, L)])
        h2_v[pl.ds(i * L, L)] = _tanh16(a + b2)

    @pl.when(do_fc)
    def _():
        pltpu.make_async_copy(bfc_hbm.at[pl.ds(wid * CPS, CPS)], bfc_v,
                              sem_w).wait()
        for r in range(N_REAL):
            pltpu.make_async_copy(
                wfc_hbm.at[r, pl.ds(wid * CPS, CPS)],
                wfc_v.at[pl.ds(r * CPS, CPS)], sem_w).wait()
        def _fc_body(n, acc):
            hb = plsc.load_gather(h2_v, [jnp.full((L,), 1, jnp.int32) + n])
            return tuple(
                acc[c] + hb * wfc_v[pl.ds(n * CPS + c * L, L)]
                for c in range(CV))

        acc0 = tuple(bfc_v[pl.ds(c * L, L)] for c in range(CV))
        acc = lax.fori_loop(0, N_REAL, _fc_body, acc0, unroll=False)
        for c in range(CV):
            out_v[pl.ds(c * L, L)] = acc[c]
        pltpu.sync_copy(out_v, out_hbm.at[pl.ds(wid * CPS, CPS)])


def kernel(feature, edge_index, W1, b1, W2, b2, Wfc, bfc):
    sf = jnp.concatenate([
        jnp.zeros((1,), jnp.float32), W1[0], b1, W2[:, 0], b2,
        feature[:, 0]])
    return _sc_model(edge_index.astype(jnp.int32), sf, Wfc, bfc)


# speedup vs baseline: 3.2524x; 1.0111x over previous
import functools

import jax
import jax.numpy as jnp
from jax import lax
from jax.experimental import pallas as pl
from jax.experimental.pallas import tpu as pltpu
from jax.experimental.pallas import tpu_sc as plsc

L = 16
N_REAL = 19
N_PAD = 32
E_REAL = 342
E_PAD = 352
EV = E_PAD // L
NV = N_PAD // L
ACC = L * N_PAD
COLS = 1600
CPS = 64
CV = CPS // L
NFC = COLS // CPS

_MESH = plsc.VectorSubcoreMesh(core_axis_name="c", subcore_axis_name="s")

_LOG2E2 = 2.885390081777927
_EXP2_C = (1.0, 0.6931471805599453, 0.2402265069591007, 0.05550410866482158,
           0.009618129107628477, 0.0013333558146428443, 0.00015403530393381608,
           1.525273380405984e-05, 1.3215486790144307e-06)


def _tanh16(v):
    t = jnp.clip(v * _LOG2E2, -30.0, 30.0)
    k = t.astype(jnp.int32)
    f = t - k.astype(jnp.float32)
    p = jnp.full((L,), _EXP2_C[8], jnp.float32)
    for i in range(7, -1, -1):
        p = p * f + _EXP2_C[i]
    e = p * plsc.bitcast((k + 127) << 23, jnp.float32)
    return (e - 1.0) / (e + 1.0)


def _rsqrt16(v):
    i = plsc.bitcast(v, jnp.int32)
    magic = jnp.full((L,), 0x5F3759DF, jnp.int32)
    y = plsc.bitcast(magic - (i >> 1), jnp.float32)
    half = 0.5 * v
    for _ in range(3):
        y = y * (1.5 - half * y * y)
    return y


def _bcast(ref, j):
    return plsc.load_gather(ref, [jnp.full((L,), j, jnp.int32)])


def _zero_acc(acc_v):
    z = jnp.zeros((L,), jnp.float32)

    def _zbody(i, carry):
        acc_v[pl.ds(i * L, L)] = z
        return carry

    lax.fori_loop(0, ACC // L, _zbody, 0, unroll=False)


def _edge_pass(srcd_v, dst_v, fs_v, acc_v, iota, ione):
    def _body(e, carry):
        s = srcd_v[pl.ds(e * L, L)]
        m = plsc.load_gather(fs_v, [s + ione])
        plsc.addupdate_scatter(acc_v, [dst_v[pl.ds(e * L, L)]], m)
        return carry

    lax.fori_loop(0, EV - 1, _body, 0, unroll=False)
    e = EV - 1
    s = plsc.load_gather(srcd_v, [jnp.minimum(iota + e * L, E_REAL - 1)])
    s = jnp.where(iota < (E_REAL - e * L), s, 0)
    m = plsc.load_gather(fs_v, [s + ione])
    plsc.addupdate_scatter(acc_v, [dst_v[pl.ds(e * L, L)]], m)


def _reduce_acc(acc_v, out_ref):
    for i in range(NV):
        def _rbody(l, ss):
            a, b = ss
            return (a + acc_v[pl.ds((2 * l) * N_PAD + i * L, L)],
                    b + acc_v[pl.ds((2 * l + 1) * N_PAD + i * L, L)])

        z = jnp.zeros((L,), jnp.float32)
        a, b = lax.fori_loop(0, L // 2, _rbody, (z, z), unroll=False)
        out_ref[pl.ds(i * L, L)] = a + b


@functools.partial(
    pl.kernel,
    out_type=jax.ShapeDtypeStruct((COLS,), jnp.float32),
    mesh=_MESH,
    compiler_params=pltpu.CompilerParams(needs_layout_passes=False),
    scratch_types=[
        pltpu.VMEM((E_REAL,), jnp.int32),
        pltpu.VMEM((E_REAL,), jnp.int32),
        pltpu.VMEM((E_PAD,), jnp.int32),
        pltpu.VMEM((30,), jnp.float32),
        pltpu.VMEM((N_PAD,), jnp.float32),
        pltpu.VMEM((N_PAD,), jnp.float32),
        pltpu.VMEM((N_PAD,), jnp.float32),
        pltpu.VMEM((ACC,), jnp.float32),
        pltpu.VMEM((N_PAD,), jnp.float32),
        pltpu.VMEM((N_PAD,), jnp.float32),
        pltpu.VMEM((N_REAL * CPS,), jnp.float32),
        pltpu.VMEM((CPS,), jnp.float32),
        pltpu.VMEM((CPS,), jnp.float32),
        pltpu.SemaphoreType.DMA,
        pltpu.SemaphoreType.DMA,
        pltpu.SemaphoreType.DMA,
    ],
)
def _sc_model(ei_hbm, sf_hbm, wfc_hbm, bfc_hbm, out_hbm,
              srcd_v, dstraw_v, dst_v, sf_v, dinv_v, fs_v, hpre_v,
              acc_v, agg_v, h2_v, wfc_v, bfc_v, out_v,
              sem_e, sem_p, sem_w):
    wid = lax.axis_index("s") * _MESH.num_cores + lax.axis_index("c")
    do_fc = wid < NFC

    cp_s0 = pltpu.async_copy(ei_hbm.at[0], srcd_v, sem_e)
    cp_d0 = pltpu.async_copy(ei_hbm.at[1], dstraw_v, sem_e)
    cp_f = pltpu.async_copy(sf_hbm, sf_v, sem_p)

    @pl.when(do_fc)
    def _():
        pltpu.async_copy(bfc_hbm.at[pl.ds(wid * CPS, CPS)], bfc_v, sem_w)

        def _wbody(r, carry):
            pltpu.async_copy(wfc_hbm.at[r, pl.ds(wid * CPS, CPS)],
                             wfc_v.at[pl.ds(r * CPS, CPS)], sem_w)
            return carry

        lax.fori_loop(0, N_REAL, _wbody, 0, unroll=False)

    ones = jnp.ones((L,), jnp.float32)
    ione = jnp.full((L,), 1, jnp.int32)
    iota = lax.iota(jnp.int32, L)
    lane32 = iota * N_PAD

    _zero_acc(acc_v)
    cp_s0.wait()
    cp_d0.wait()

    def _repack_body(e, carry):
        idx = iota + e * L
        d = plsc.load_gather(dstraw_v, [idx]) + ione
        doff = d + lane32
        dst_v[pl.ds(e * L, L)] = doff
        plsc.addupdate_scatter(acc_v, [doff], ones)
        return carry

    lax.fori_loop(0, EV - 1, _repack_body, 0, unroll=False)
    e = EV - 1
    idx = iota + e * L
    valid = iota < (E_REAL - e * L)
    d = plsc.load_gather(dstraw_v, [jnp.minimum(idx, E_REAL - 1)])
    d = jnp.where(valid, d + ione, 0)
    doff = d + lane32
    dst_v[pl.ds(e * L, L)] = doff
    plsc.addupdate_scatter(acc_v, [doff], ones)
    _reduce_acc(acc_v, dinv_v)
    for i in range(NV):
        d = dinv_v[pl.ds(i * L, L)] + ones
        dinv_v[pl.ds(i * L, L)] = _rsqrt16(d)

    ifeat = jnp.full((L,), 11, jnp.int32)
    cp_f.wait()

    for i in range(NV):
        dv = dinv_v[pl.ds(i * L, L)]
        fshift = plsc.load_gather(
            sf_v, [jnp.clip(iota + (i * L - 1), 0, N_REAL - 1) + ifeat])
        fshift = jnp.where(
            (iota + i * L >= 1) & (iota + i * L <= N_REAL), fshift, 0.0)
        fs_v[pl.ds(i * L, L)] = fshift * dv

    _zero_acc(acc_v)
    _edge_pass(srcd_v, dst_v, fs_v, acc_v, iota, ione)
    _reduce_acc(acc_v, agg_v)
    for i in range(NV):
        dv = dinv_v[pl.ds(i * L, L)]
        agg_v[pl.ds(i * L, L)] = dv * (agg_v[pl.ds(i * L, L)] +
                                       fs_v[pl.ds(i * L, L)])

    for i in range(NV):
        a = agg_v[pl.ds(i * L, L)]
        acc = jnp.zeros((L,), jnp.float32)
        for k in range(3):
            w1k = _bcast(sf_v, 1 + k)
            b1k = _bcast(sf_v, 4 + k)
            w2k = _bcast(sf_v, 7 + k)
            acc = acc + _tanh16(w1k * a + b1k) * w2k
        hpre_v[pl.ds(i * L, L)] = acc

    for i in range(NV):
        fs_v[pl.ds(i * L, L)] = (hpre_v[pl.ds(i * L, L)] *
                                 dinv_v[pl.ds(i * L, L)])
    _zero_acc(acc_v)
    _edge_pass(srcd_v, dst_v, fs_v, acc_v, iota, ione)
    _reduce_acc(acc_v, agg_v)
    b2 = _bcast(sf_v, 10)
    for i in range(NV):
        dv = dinv_v[pl.ds(i * L, L)]
        a = dv * (agg_v[pl.ds(i * L, L)] + fs_v[pl.ds(i * L, L)])
        h2_v[pl.ds(i * L, L)] = _tanh16(a + b2)

    @pl.when(do_fc)
    def _():
        pltpu.make_async_copy(bfc_hbm.at[pl.ds(wid * CPS, CPS)], bfc_v,
                              sem_w).wait()

        def _dbody(r, carry):
            pltpu.make_async_copy(
                wfc_hbm.at[r, pl.ds(wid * CPS, CPS)],
                wfc_v.at[pl.ds(r * CPS, CPS)], sem_w).wait()
            return carry

        lax.fori_loop(0, N_REAL, _dbody, 0, unroll=False)
        def _fc_body(n, acc):
            hb = plsc.load_gather(h2_v, [jnp.full((L,), 1, jnp.int32) + n])
            return tuple(
                acc[c] + hb * wfc_v[pl.ds(n * CPS + c * L, L)]
                for c in range(CV))

        acc0 = tuple(bfc_v[pl.ds(c * L, L)] for c in range(CV))
        acc = lax.fori_loop(0, N_REAL, _fc_body, acc0, unroll=False)
        for c in range(CV):
            out_v[pl.ds(c * L, L)] = acc[c]
        pltpu.sync_copy(out_v, out_hbm.at[pl.ds(wid * CPS, CPS)])


def kernel(feature, edge_index, W1, b1, W2, b2, Wfc, bfc):
    sf = jnp.concatenate([
        jnp.zeros((1,), jnp.float32), W1[0], b1, W2[:, 0], b2,
        feature[:, 0]])
    return _sc_model(edge_index.astype(jnp.int32), sf, Wfc, bfc)


# speedup vs baseline: 3.3839x; 1.0404x over previous
import functools

import jax
import jax.numpy as jnp
from jax import lax
from jax.experimental import pallas as pl
from jax.experimental.pallas import tpu as pltpu
from jax.experimental.pallas import tpu_sc as plsc

L = 16
N_REAL = 19
N_PAD = 32
E_REAL = 342
E_PAD = 352
EV = E_PAD // L
NV = N_PAD // L
ACC = L * N_PAD
COLS = 1600
CPS = 64
CV = CPS // L
NFC = COLS // CPS

_MESH = plsc.VectorSubcoreMesh(core_axis_name="c", subcore_axis_name="s")

_LOG2E2 = 2.885390081777927
_EXP2_C = (1.0, 0.6931471805599453, 0.2402265069591007, 0.05550410866482158,
           0.009618129107628477, 0.0013333558146428443, 0.00015403530393381608,
           1.525273380405984e-05, 1.3215486790144307e-06)


def _tanh16(v):
    t = jnp.clip(v * _LOG2E2, -30.0, 30.0)
    k = t.astype(jnp.int32)
    f = t - k.astype(jnp.float32)
    p = jnp.full((L,), _EXP2_C[8], jnp.float32)
    for i in range(7, -1, -1):
        p = p * f + _EXP2_C[i]
    e = p * plsc.bitcast((k + 127) << 23, jnp.float32)
    return (e - 1.0) / (e + 1.0)


def _rsqrt16(v):
    i = plsc.bitcast(v, jnp.int32)
    magic = jnp.full((L,), 0x5F3759DF, jnp.int32)
    y = plsc.bitcast(magic - (i >> 1), jnp.float32)
    half = 0.5 * v
    for _ in range(3):
        y = y * (1.5 - half * y * y)
    return y


def _bcast(ref, j):
    return plsc.load_gather(ref, [jnp.full((L,), j, jnp.int32)])


def _zero_acc(acc_v):
    z = jnp.zeros((L,), jnp.float32)

    @functools.partial(plsc.parallel_loop, 0, ACC // L, unroll=8)
    def _zbody(i):
        acc_v[pl.ds(i * L, L)] = z


def _edge_pass(srcd_v, dst_v, fs_v, acc_v, iota, ione):
    @functools.partial(plsc.parallel_loop, 0, EV - 1, unroll=7)
    def _body(e):
        s = srcd_v[pl.ds(e * L, L)]
        m = plsc.load_gather(fs_v, [s + ione])
        plsc.addupdate_scatter(acc_v, [dst_v[pl.ds(e * L, L)]], m)
    e = EV - 1
    s = plsc.load_gather(srcd_v, [jnp.minimum(iota + e * L, E_REAL - 1)])
    s = jnp.where(iota < (E_REAL - e * L), s, 0)
    m = plsc.load_gather(fs_v, [s + ione])
    plsc.addupdate_scatter(acc_v, [dst_v[pl.ds(e * L, L)]], m)


def _reduce_acc(acc_v, out_ref):
    for i in range(NV):
        def _rbody(l, ss):
            a, b = ss
            return (a + acc_v[pl.ds((2 * l) * N_PAD + i * L, L)],
                    b + acc_v[pl.ds((2 * l + 1) * N_PAD + i * L, L)])

        z = jnp.zeros((L,), jnp.float32)
        a, b = lax.fori_loop(0, L // 2, _rbody, (z, z), unroll=False)
        out_ref[pl.ds(i * L, L)] = a + b


@functools.partial(
    pl.kernel,
    out_type=jax.ShapeDtypeStruct((COLS,), jnp.float32),
    mesh=_MESH,
    compiler_params=pltpu.CompilerParams(needs_layout_passes=False),
    scratch_types=[
        pltpu.VMEM((E_REAL,), jnp.int32),
        pltpu.VMEM((E_REAL,), jnp.int32),
        pltpu.VMEM((E_PAD,), jnp.int32),
        pltpu.VMEM((30,), jnp.float32),
        pltpu.VMEM((N_PAD,), jnp.float32),
        pltpu.VMEM((N_PAD,), jnp.float32),
        pltpu.VMEM((N_PAD,), jnp.float32),
        pltpu.VMEM((ACC,), jnp.float32),
        pltpu.VMEM((N_PAD,), jnp.float32),
        pltpu.VMEM((N_PAD,), jnp.float32),
        pltpu.VMEM((N_REAL * CPS,), jnp.float32),
        pltpu.VMEM((CPS,), jnp.float32),
        pltpu.VMEM((CPS,), jnp.float32),
        pltpu.SemaphoreType.DMA,
        pltpu.SemaphoreType.DMA,
        pltpu.SemaphoreType.DMA,
    ],
)
def _sc_model(ei_hbm, sf_hbm, wfc_hbm, bfc_hbm, out_hbm,
              srcd_v, dstraw_v, dst_v, sf_v, dinv_v, fs_v, hpre_v,
              acc_v, agg_v, h2_v, wfc_v, bfc_v, out_v,
              sem_e, sem_p, sem_w):
    wid = lax.axis_index("s") * _MESH.num_cores + lax.axis_index("c")
    do_fc = wid < NFC

    cp_s0 = pltpu.async_copy(ei_hbm.at[0], srcd_v, sem_e)
    cp_d0 = pltpu.async_copy(ei_hbm.at[1], dstraw_v, sem_e)
    cp_f = pltpu.async_copy(sf_hbm, sf_v, sem_p)

    @pl.when(do_fc)
    def _():
        pltpu.async_copy(bfc_hbm.at[pl.ds(wid * CPS, CPS)], bfc_v, sem_w)

        def _wbody(r, carry):
            pltpu.async_copy(wfc_hbm.at[r, pl.ds(wid * CPS, CPS)],
                             wfc_v.at[pl.ds(r * CPS, CPS)], sem_w)
            return carry

        lax.fori_loop(0, N_REAL, _wbody, 0, unroll=False)

    ones = jnp.ones((L,), jnp.float32)
    ione = jnp.full((L,), 1, jnp.int32)
    iota = lax.iota(jnp.int32, L)
    lane32 = iota * N_PAD

    _zero_acc(acc_v)
    cp_s0.wait()
    cp_d0.wait()

    @functools.partial(plsc.parallel_loop, 0, EV - 1, unroll=7)
    def _repack_body(e):
        idx = iota + e * L
        d = plsc.load_gather(dstraw_v, [idx]) + ione
        doff = d + lane32
        dst_v[pl.ds(e * L, L)] = doff
        plsc.addupdate_scatter(acc_v, [doff], ones)
    e = EV - 1
    idx = iota + e * L
    valid = iota < (E_REAL - e * L)
    d = plsc.load_gather(dstraw_v, [jnp.minimum(idx, E_REAL - 1)])
    d = jnp.where(valid, d + ione, 0)
    doff = d + lane32
    dst_v[pl.ds(e * L, L)] = doff
    plsc.addupdate_scatter(acc_v, [doff], ones)
    _reduce_acc(acc_v, dinv_v)
    for i in range(NV):
        d = dinv_v[pl.ds(i * L, L)] + ones
        dinv_v[pl.ds(i * L, L)] = _rsqrt16(d)

    ifeat = jnp.full((L,), 11, jnp.int32)
    cp_f.wait()

    for i in range(NV):
        dv = dinv_v[pl.ds(i * L, L)]
        fshift = plsc.load_gather(
            sf_v, [jnp.clip(iota + (i * L - 1), 0, N_REAL - 1) + ifeat])
        fshift = jnp.where(
            (iota + i * L >= 1) & (iota + i * L <= N_REAL), fshift, 0.0)
        fs_v[pl.ds(i * L, L)] = fshift * dv

    _zero_acc(acc_v)
    _edge_pass(srcd_v, dst_v, fs_v, acc_v, iota, ione)
    _reduce_acc(acc_v, agg_v)
    for i in range(NV):
        dv = dinv_v[pl.ds(i * L, L)]
        agg_v[pl.ds(i * L, L)] = dv * (agg_v[pl.ds(i * L, L)] +
                                       fs_v[pl.ds(i * L, L)])

    for i in range(NV):
        a = agg_v[pl.ds(i * L, L)]
        acc = jnp.zeros((L,), jnp.float32)
        for k in range(3):
            w1k = _bcast(sf_v, 1 + k)
            b1k = _bcast(sf_v, 4 + k)
            w2k = _bcast(sf_v, 7 + k)
            acc = acc + _tanh16(w1k * a + b1k) * w2k
        hpre_v[pl.ds(i * L, L)] = acc

    for i in range(NV):
        fs_v[pl.ds(i * L, L)] = (hpre_v[pl.ds(i * L, L)] *
                                 dinv_v[pl.ds(i * L, L)])
    _zero_acc(acc_v)
    _edge_pass(srcd_v, dst_v, fs_v, acc_v, iota, ione)
    _reduce_acc(acc_v, agg_v)
    b2 = _bcast(sf_v, 10)
    for i in range(NV):
        dv = dinv_v[pl.ds(i * L, L)]
        a = dv * (agg_v[pl.ds(i * L, L)] + fs_v[pl.ds(i * L, L)])
        h2_v[pl.ds(i * L, L)] = _tanh16(a + b2)

    @pl.when(do_fc)
    def _():
        pltpu.make_async_copy(bfc_hbm.at[pl.ds(wid * CPS, CPS)], bfc_v,
                              sem_w).wait()

        def _dbody(r, carry):
            pltpu.make_async_copy(
                wfc_hbm.at[r, pl.ds(wid * CPS, CPS)],
                wfc_v.at[pl.ds(r * CPS, CPS)], sem_w).wait()
            return carry

        lax.fori_loop(0, N_REAL, _dbody, 0, unroll=False)
        def _fc_body(n, acc):
            hb = plsc.load_gather(h2_v, [jnp.full((L,), 1, jnp.int32) + n])
            return tuple(
                acc[c] + hb * wfc_v[pl.ds(n * CPS + c * L, L)]
                for c in range(CV))

        acc0 = tuple(bfc_v[pl.ds(c * L, L)] for c in range(CV))
        acc = lax.fori_loop(0, N_REAL, _fc_body, acc0, unroll=False)
        for c in range(CV):
            out_v[pl.ds(c * L, L)] = acc[c]
        pltpu.sync_copy(out_v, out_hbm.at[pl.ds(wid * CPS, CPS)])


def kernel(feature, edge_index, W1, b1, W2, b2, Wfc, bfc):
    sf = jnp.concatenate([
        jnp.zeros((1,), jnp.float32), W1[0], b1, W2[:, 0], b2,
        feature[:, 0]])
    return _sc_model(edge_index.astype(jnp.int32), sf, Wfc, bfc)
